# baseline (speedup 1.0000x reference)
import jax
import jax.numpy as jnp
from jax.experimental import pallas as pl
from jax.experimental.pallas import tpu as pltpu


def _nt_dot(x, w):
    return jax.lax.dot_general(
        x, w, (((1,), (1,)), ((), ())), preferred_element_type=jnp.float32
    )


def _mlp_kernel(x_ref, w0_ref, b0_ref, w1_ref, b1_ref, w2_ref, b2_ref, o_ref):
    zero = jnp.bfloat16(0)
    w0 = w0_ref[0].astype(jnp.bfloat16)
    w1 = w1_ref[0].astype(jnp.bfloat16)
    w2 = w2_ref[0].astype(jnp.bfloat16)
    S = x_ref.shape[1]
    C = 512
    for c in range(S // C):
        x = x_ref[0, c * C:(c + 1) * C, :].astype(jnp.bfloat16)
        h = jnp.maximum((_nt_dot(x, w0) + b0_ref[0]).astype(jnp.bfloat16), zero)
        h = jnp.maximum((_nt_dot(h, w1) + b1_ref[0]).astype(jnp.bfloat16), zero)
        o_ref[0, c * C:(c + 1) * C, :] = _nt_dot(h, w2) + b2_ref[0]


def kernel(query, W0, b0, W1, b1, W2, b2):
    B, S, D_IN = query.shape
    D_H = W0.shape[1]
    D_OUT = W2.shape[1]

    b0r = b0[:, None, :]
    b1r = b1[:, None, :]
    b2r = b2[:, None, :]

    return pl.pallas_call(
        _mlp_kernel,
        out_shape=jax.ShapeDtypeStruct((B, S, D_OUT), query.dtype),
        grid=(B,),
        in_specs=[
            pl.BlockSpec((1, S, D_IN), lambda b: (b, 0, 0)),
            pl.BlockSpec((1, D_H, D_IN), lambda b: (b, 0, 0)),
            pl.BlockSpec((1, 1, D_H), lambda b: (b, 0, 0)),
            pl.BlockSpec((1, D_H, D_H), lambda b: (b, 0, 0)),
            pl.BlockSpec((1, 1, D_H), lambda b: (b, 0, 0)),
            pl.BlockSpec((1, D_OUT, D_H), lambda b: (b, 0, 0)),
            pl.BlockSpec((1, 1, D_OUT), lambda b: (b, 0, 0)),
        ],
        out_specs=pl.BlockSpec((1, S, D_OUT), lambda b: (b, 0, 0)),
        compiler_params=pltpu.CompilerParams(
            dimension_semantics=("parallel",),
            vmem_limit_bytes=62 * 1024 * 1024,
        ),
        name="ltm_mlp",
    )(query, W0, b0r, W1, b1r, W2, b2r)


# speedup vs baseline: 1.0038x; 1.0038x over previous
import jax
import jax.numpy as jnp
from jax.experimental import pallas as pl
from jax.experimental.pallas import tpu as pltpu


def _nt_dot(x, w):
    return jax.lax.dot_general(
        x, w, (((1,), (1,)), ((), ())), preferred_element_type=jnp.float32
    )


def _mlp_kernel(x_ref, w0_ref, b0_ref, w1_ref, b1_ref, w2_ref, b2_ref, o_ref):
    zero = jnp.bfloat16(0)
    w0 = w0_ref[0].astype(jnp.bfloat16)
    w1 = w1_ref[0].astype(jnp.bfloat16)
    w2 = w2_ref[0].astype(jnp.bfloat16)
    S = x_ref.shape[1]
    C = 1024
    for c in range(S // C):
        x = x_ref[0, c * C:(c + 1) * C, :].astype(jnp.bfloat16)
        h = jnp.maximum((_nt_dot(x, w0) + b0_ref[0]).astype(jnp.bfloat16), zero)
        h = jnp.maximum((_nt_dot(h, w1) + b1_ref[0]).astype(jnp.bfloat16), zero)
        o_ref[0, c * C:(c + 1) * C, :] = _nt_dot(h, w2) + b2_ref[0]


def kernel(query, W0, b0, W1, b1, W2, b2):
    B, S, D_IN = query.shape
    D_H = W0.shape[1]
    D_OUT = W2.shape[1]

    b0r = b0[:, None, :]
    b1r = b1[:, None, :]
    b2r = b2[:, None, :]

    return pl.pallas_call(
        _mlp_kernel,
        out_shape=jax.ShapeDtypeStruct((B, S, D_OUT), query.dtype),
        grid=(B,),
        in_specs=[
            pl.BlockSpec((1, S, D_IN), lambda b: (b, 0, 0)),
            pl.BlockSpec((1, D_H, D_IN), lambda b: (b, 0, 0)),
            pl.BlockSpec((1, 1, D_H), lambda b: (b, 0, 0)),
            pl.BlockSpec((1, D_H, D_H), lambda b: (b, 0, 0)),
            pl.BlockSpec((1, 1, D_H), lambda b: (b, 0, 0)),
            pl.BlockSpec((1, D_OUT, D_H), lambda b: (b, 0, 0)),
            pl.BlockSpec((1, 1, D_OUT), lambda b: (b, 0, 0)),
        ],
        out_specs=pl.BlockSpec((1, S, D_OUT), lambda b: (b, 0, 0)),
        compiler_params=pltpu.CompilerParams(
            dimension_semantics=("parallel",),
            vmem_limit_bytes=62 * 1024 * 1024,
        ),
        name="ltm_mlp",
    )(query, W0, b0r, W1, b1r, W2, b2r)
